# baseline (speedup 1.0000x reference)
import functools

import jax
import jax.numpy as jnp
from jax import lax
from jax.experimental import pallas as pl
from jax.experimental.pallas import tpu as pltpu
from jax.experimental.pallas import tpu_sc as plsc

_L = 16
_VB = 147456


def _make_tc_project(C, D, V, nv):
    vb8 = _VB // 8

    def body(u_ref, i_ref, mm_ref, mel_ref, wf_ref, dummy_ref, s_ref):
        wf = wf_ref[...]
        wu = wf[:D, 0]
        wi = (mm_ref[...] @ (mel_ref[...] @ wf[D:]))[:, 0]
        u = u_ref[0]
        i = i_ref[0]
        s_ref[0] = jnp.sum(u * wu[:, None], axis=0).reshape(8, vb8)
        s_ref[1] = jnp.sum(i * wi[:, None], axis=0).reshape(8, vb8)

    grid = (C, nv)
    return pl.pallas_call(
        body,
        grid=grid,
        in_specs=[
            pl.BlockSpec((1, D, _VB), lambda c, j: (c, 0, j)),
            pl.BlockSpec((1, D, _VB), lambda c, j: (c, 0, j)),
            pl.BlockSpec((D, 2 * D), lambda c, j: (0, 0)),
            pl.BlockSpec((2 * D, D), lambda c, j: (0, 0)),
            pl.BlockSpec((2 * D, 1), lambda c, j: (0, 0)),
            pl.BlockSpec(memory_space=pltpu.MemorySpace.HBM),
        ],
        out_specs=pl.BlockSpec((2, 8, vb8), lambda c, j: (0, c * nv + j, 0)),
        out_shape=jax.ShapeDtypeStruct((2, C * nv * 8, vb8), jnp.float32),
        input_output_aliases={5: 0},
        compiler_params=pltpu.CompilerParams(vmem_limit_bytes=50 * 1024 * 1024),
    )


def _make_sc_gather(CB):
    info = plsc.get_sparse_core_info()
    NC, NS = info.num_cores, info.num_subcores
    NW = NC * NS
    npw = CB // NW
    nchunk = npw // 128
    nblk = npw // _L
    B = CB // 2
    wpc = NW // 2
    assert CB % NW == 0 and npw % 128 == 0 and B % npw == 0

    mesh = plsc.VectorSubcoreMesh(core_axis_name="c", subcore_axis_name="s")

    @functools.partial(
        pl.kernel,
        mesh=mesh,
        out_type=[jax.ShapeDtypeStruct((B,), jnp.float32),
                  jax.ShapeDtypeStruct((B,), jnp.float32)],
        scratch_types=[
            pltpu.VMEM((nchunk, 128), jnp.int32),
            pltpu.VMEM((nchunk, 128), jnp.int32),
            pltpu.VMEM((npw,), jnp.float32),
            pltpu.VMEM((npw,), jnp.float32),
            pltpu.VMEM((npw,), jnp.float32),
            pltpu.VMEM((_L,), jnp.float32),
            pltpu.SemaphoreType.DMA,
        ],
        compiler_params=pltpu.CompilerParams(
            needs_layout_passes=False, use_tc_tiling_on_sc=False),
    )
    def sc_kernel(uidx_hbm, iidx_hbm, s_hbm, bias_hbm, out0_hbm, out1_hbm,
                  uidx_v, iidx_v, sgu_v, sgi_v, out_v, bias_v, sem):
        wid = lax.axis_index("s") * NC + lax.axis_index("c")

        pltpu.sync_copy(uidx_hbm.at[pl.ds(wid * nchunk, nchunk)], uidx_v)
        pltpu.sync_copy(iidx_hbm.at[pl.ds(wid * nchunk, nchunk)], iidx_v)

        copies = []
        for j in range(nchunk):
            copies.append(pltpu.async_copy(
                s_hbm.at[uidx_v.at[j]], sgu_v.at[pl.ds(j * 128, 128)], sem))
        for j in range(nchunk):
            copies.append(pltpu.async_copy(
                s_hbm.at[iidx_v.at[j]], sgi_v.at[pl.ds(j * 128, 128)], sem))

        pltpu.sync_copy(bias_hbm, bias_v)
        b_s = bias_v[pl.ds(0, _L)][0]

        for cp in copies:
            cp.wait()

        def blk(k, carry):
            x = sgu_v[pl.ds(k * _L, _L)] + sgi_v[pl.ds(k * _L, _L)] + b_s
            out_v[pl.ds(k * _L, _L)] = 1.0 / (1.0 + jnp.exp(-x))
            return carry
        lax.fori_loop(0, nblk, blk, 0)

        @pl.when(wid < wpc)
        def _():
            pltpu.sync_copy(out_v, out0_hbm.at[pl.ds(wid * npw, npw)])

        @pl.when(wid >= wpc)
        def _():
            pltpu.sync_copy(out_v, out1_hbm.at[pl.ds((wid - wpc) * npw, npw)])

    return sc_kernel


def kernel(inputs, user_tables, item_tables, meta_memory, meta_emb_layer,
           w_final, b_final):
    C, B, _ = inputs.shape
    _, V, D = user_tables.shape
    CB = C * B
    nv = (V + _VB - 1) // _VB
    vpad = nv * _VB

    utab_t = user_tables.transpose(0, 2, 1)
    itab_t = item_tables.transpose(0, 2, 1)
    ids = inputs.astype(jnp.int32)
    offs = (jnp.arange(C, dtype=jnp.int32) * vpad)[:, None]
    uidx = (ids[:, :, 0] + offs).reshape(CB // 128, 128)
    iidx = (ids[:, :, 1] + offs + C * vpad).reshape(CB // 128, 128)

    tc = _make_tc_project(C, D, V, nv)
    dummy = jnp.zeros((2, C * nv * 8, _VB // 8), jnp.float32)
    s2 = tc(utab_t, itab_t, meta_memory, meta_emb_layer,
            w_final.astype(jnp.float32), dummy)
    s = s2.reshape(2 * C * vpad)

    bias = jnp.broadcast_to(b_final.astype(jnp.float32), (_L,))

    sc = _make_sc_gather(CB)
    out0, out1 = sc(uidx, iidx, s, bias)
    return (out0, out1)


# speedup vs baseline: 1.0547x; 1.0547x over previous
import functools

import jax
import jax.numpy as jnp
from jax import lax
from jax.experimental import pallas as pl
from jax.experimental.pallas import tpu as pltpu
from jax.experimental.pallas import tpu_sc as plsc

_L = 16
_VB = 147456


def _make_tc_project(C, D, V, nv):
    vb8 = _VB // 8

    def body(u_ref, i_ref, mm_ref, mel_ref, wf_ref, s_ref):
        wf = wf_ref[...]
        wu = wf[:D, 0]
        wi = (mm_ref[...] @ (mel_ref[...] @ wf[D:]))[:, 0]
        u = u_ref[0]
        i = i_ref[0]
        s_ref[0] = jnp.sum(u * wu[:, None], axis=0).reshape(8, vb8)
        s_ref[1] = jnp.sum(i * wi[:, None], axis=0).reshape(8, vb8)

    grid = (C, nv)
    return pl.pallas_call(
        body,
        grid=grid,
        in_specs=[
            pl.BlockSpec((1, D, _VB), lambda c, j: (c, 0, j)),
            pl.BlockSpec((1, D, _VB), lambda c, j: (c, 0, j)),
            pl.BlockSpec((D, 2 * D), lambda c, j: (0, 0)),
            pl.BlockSpec((2 * D, D), lambda c, j: (0, 0)),
            pl.BlockSpec((2 * D, 1), lambda c, j: (0, 0)),
        ],
        out_specs=pl.BlockSpec((2, 8, vb8), lambda c, j: (0, c * nv + j, 0)),
        out_shape=jax.ShapeDtypeStruct((2, C * nv * 8, vb8), jnp.float32),
        compiler_params=pltpu.CompilerParams(vmem_limit_bytes=50 * 1024 * 1024),
    )


def _make_sc_gather(CB):
    info = plsc.get_sparse_core_info()
    NC, NS = info.num_cores, info.num_subcores
    NW = NC * NS
    npw = CB // NW
    nchunk = npw // 128
    nblk = npw // _L
    B = CB // 2
    wpc = NW // 2
    assert CB % NW == 0 and npw % 128 == 0 and B % npw == 0

    mesh = plsc.VectorSubcoreMesh(core_axis_name="c", subcore_axis_name="s")

    @functools.partial(
        pl.kernel,
        mesh=mesh,
        out_type=[jax.ShapeDtypeStruct((B,), jnp.float32),
                  jax.ShapeDtypeStruct((B,), jnp.float32)],
        scratch_types=[
            pltpu.VMEM((nchunk, 128), jnp.int32),
            pltpu.VMEM((nchunk, 128), jnp.int32),
            pltpu.VMEM((npw,), jnp.float32),
            pltpu.VMEM((npw,), jnp.float32),
            pltpu.VMEM((npw,), jnp.float32),
            pltpu.VMEM((_L,), jnp.float32),
            pltpu.SemaphoreType.DMA,
        ],
        compiler_params=pltpu.CompilerParams(
            needs_layout_passes=False, use_tc_tiling_on_sc=False),
    )
    def sc_kernel(uidx_hbm, iidx_hbm, s_hbm, bias_hbm, out0_hbm, out1_hbm,
                  uidx_v, iidx_v, sgu_v, sgi_v, out_v, bias_v, sem):
        wid = lax.axis_index("s") * NC + lax.axis_index("c")

        pltpu.sync_copy(uidx_hbm.at[pl.ds(wid * nchunk, nchunk)], uidx_v)
        pltpu.sync_copy(iidx_hbm.at[pl.ds(wid * nchunk, nchunk)], iidx_v)

        copies = []
        for j in range(nchunk):
            copies.append(pltpu.async_copy(
                s_hbm.at[uidx_v.at[j]], sgu_v.at[pl.ds(j * 128, 128)], sem))
        for j in range(nchunk):
            copies.append(pltpu.async_copy(
                s_hbm.at[iidx_v.at[j]], sgi_v.at[pl.ds(j * 128, 128)], sem))

        pltpu.sync_copy(bias_hbm, bias_v)
        b_s = bias_v[pl.ds(0, _L)][0]

        for cp in copies:
            cp.wait()

        def blk(k, carry):
            x = sgu_v[pl.ds(k * _L, _L)] + sgi_v[pl.ds(k * _L, _L)] + b_s
            out_v[pl.ds(k * _L, _L)] = 1.0 / (1.0 + jnp.exp(-x))
            return carry
        lax.fori_loop(0, nblk, blk, 0)

        @pl.when(wid < wpc)
        def _():
            pltpu.sync_copy(out_v, out0_hbm.at[pl.ds(wid * npw, npw)])

        @pl.when(wid >= wpc)
        def _():
            pltpu.sync_copy(out_v, out1_hbm.at[pl.ds((wid - wpc) * npw, npw)])

    return sc_kernel


def kernel(inputs, user_tables, item_tables, meta_memory, meta_emb_layer,
           w_final, b_final):
    C, B, _ = inputs.shape
    _, V, D = user_tables.shape
    CB = C * B
    nv = (V + _VB - 1) // _VB
    vpad = nv * _VB

    utab_t = user_tables.transpose(0, 2, 1)
    itab_t = item_tables.transpose(0, 2, 1)
    ids = inputs.astype(jnp.int32)
    offs = (jnp.arange(C, dtype=jnp.int32) * vpad)[:, None]
    uidx = (ids[:, :, 0] + offs).reshape(CB // 128, 128)
    iidx = (ids[:, :, 1] + offs + C * vpad).reshape(CB // 128, 128)

    tc = _make_tc_project(C, D, V, nv)
    s2 = tc(utab_t, itab_t, meta_memory, meta_emb_layer,
            w_final.astype(jnp.float32))
    s = s2.reshape(2 * C * vpad)

    bias = jnp.broadcast_to(b_final.astype(jnp.float32), (_L,))

    sc = _make_sc_gather(CB)
    out0, out1 = sc(uidx, iidx, s, bias)
    return (out0, out1)
